# baseline (speedup 1.0000x reference)
import jax
import jax.numpy as jnp
from jax import lax
from jax.experimental import pallas as pl
from jax.experimental.pallas import tpu as pltpu
from jax.experimental.pallas import tpu_sc as plsc

N_NODES = 10000
D_FEAT = 256
N_EDGES = 160000

NUM_CORES = 2
NUM_SUBCORES = 16
NUM_WORKERS = NUM_CORES * NUM_SUBCORES
CHUNK = 40
NUM_CHUNKS = 130
EDGES_PER_WORKER = CHUNK * NUM_CHUNKS
E_PAD = EDGES_PER_WORKER * NUM_WORKERS
LANES = 16


def _sc_body(emb_hbm, src_hbm, dst_hbm, out_hbm,
             idx_s_v, idx_t_v, rows_s_v, rows_t_v, rows_s1, rows_t1, out_v,
             sem, sem1):
    wid = lax.axis_index("s") * NUM_CORES + lax.axis_index("c")
    base = wid * EDGES_PER_WORKER

    pltpu.sync_copy(src_hbm.at[pl.ds(base, EDGES_PER_WORKER)], idx_s_v)
    pltpu.sync_copy(dst_hbm.at[pl.ds(base, EDGES_PER_WORKER)], idx_t_v)

    def chunk_body(ci, carry):
        off = ci * CHUNK
        cp_s = pltpu.async_copy(emb_hbm.at[idx_s_v.at[pl.ds(off, CHUNK)]],
                                rows_s_v, sem)
        cp_t = pltpu.async_copy(emb_hbm.at[idx_t_v.at[pl.ds(off, CHUNK)]],
                                rows_t_v, sem)
        cp_s.wait()
        cp_t.wait()

        lane = lax.iota(jnp.int32, LANES)
        last_lane = lane == (LANES - 1)

        def edge_body(e, carry2):
            acc = rows_s_v[e, pl.ds(0, LANES)] * rows_t_v[e, pl.ds(0, LANES)]
            for j in range(1, D_FEAT // LANES):
                acc = acc + (rows_s_v[e, pl.ds(j * LANES, LANES)]
                             * rows_t_v[e, pl.ds(j * LANES, LANES)])
            tot = plsc.cumsum(acc)
            plsc.store_scatter(out_v, [jnp.full((LANES,), off + e, jnp.int32)],
                               tot, mask=last_lane)
            return carry2

        lax.fori_loop(0, CHUNK, edge_body, 0, unroll=4)
        return carry

    lax.fori_loop(0, NUM_CHUNKS, chunk_body, 0)
    pltpu.sync_copy(out_v, out_hbm.at[pl.ds(base, EDGES_PER_WORKER)])


def kernel(node_embeddings, edge_label_index):
    idx = edge_label_index.astype(jnp.int32)
    pad = jnp.zeros((2, E_PAD - N_EDGES), jnp.int32)
    idx = jnp.concatenate([idx, pad], axis=1)
    src = idx[0]
    dst = idx[1]

    mesh = plsc.VectorSubcoreMesh(core_axis_name="c", subcore_axis_name="s")
    f = pl.kernel(
        _sc_body,
        mesh=mesh,
        compiler_params=pltpu.CompilerParams(needs_layout_passes=False),
        out_type=jax.ShapeDtypeStruct((E_PAD,), jnp.float32),
        scratch_types=[
            pltpu.VMEM((EDGES_PER_WORKER,), jnp.int32),
            pltpu.VMEM((EDGES_PER_WORKER,), jnp.int32),
            pltpu.VMEM((CHUNK, D_FEAT), jnp.float32),
            pltpu.VMEM((CHUNK, D_FEAT), jnp.float32),
            pltpu.VMEM((CHUNK, D_FEAT), jnp.float32),
            pltpu.VMEM((CHUNK, D_FEAT), jnp.float32),
            pltpu.VMEM((EDGES_PER_WORKER,), jnp.float32),
            pltpu.SemaphoreType.DMA,
            pltpu.SemaphoreType.DMA,
        ],
    )
    return f(node_embeddings, src, dst)[:N_EDGES]


# speedup vs baseline: 2.1757x; 2.1757x over previous
import jax
import jax.numpy as jnp
from jax import lax
from jax.experimental import pallas as pl
from jax.experimental.pallas import tpu as pltpu
from jax.experimental.pallas import tpu_sc as plsc

N_NODES = 10000
D_FEAT = 256
N_EDGES = 160000

NUM_CORES = 2
NUM_SUBCORES = 16
NUM_WORKERS = NUM_CORES * NUM_SUBCORES
CHUNK = 40
NUM_CHUNKS = 130
EDGES_PER_WORKER = CHUNK * NUM_CHUNKS
E_PAD = EDGES_PER_WORKER * NUM_WORKERS
LANES = 16


def _sc_body(emb_hbm, src_hbm, dst_hbm, out_hbm,
             idx_s_v, idx_t_v, rows_s_v, rows_t_v, rows_s1, rows_t1, out_v,
             sem, sem1):
    wid = lax.axis_index("s") * NUM_CORES + lax.axis_index("c")
    base = wid * EDGES_PER_WORKER

    pltpu.sync_copy(src_hbm.at[pl.ds(base, EDGES_PER_WORKER)], idx_s_v)
    pltpu.sync_copy(dst_hbm.at[pl.ds(base, EDGES_PER_WORKER)], idx_t_v)

    def chunk_body(ci, carry):
        off = ci * CHUNK
        cp_s = pltpu.async_copy(emb_hbm.at[idx_s_v.at[pl.ds(off, CHUNK)]],
                                rows_s_v, sem)
        cp_t = pltpu.async_copy(emb_hbm.at[idx_t_v.at[pl.ds(off, CHUNK)]],
                                rows_t_v, sem)
        cp_s.wait()
        cp_t.wait()

        lane = lax.iota(jnp.int32, LANES)
        last_lane = lane == (LANES - 1)

        def edge_body(e, carry2):
            acc = rows_s_v[e, pl.ds(0, LANES)] * rows_t_v[e, pl.ds(0, LANES)]
            for j in range(1, D_FEAT // LANES):
                acc = acc + (rows_s_v[e, pl.ds(j * LANES, LANES)]
                             * rows_t_v[e, pl.ds(j * LANES, LANES)])
            tot = plsc.cumsum(acc)
            plsc.store_scatter(out_v, [jnp.full((LANES,), off + e, jnp.int32)],
                               tot, mask=last_lane)
            return carry2

        lax.fori_loop(0, CHUNK, edge_body, 0, unroll=4)
        return carry

    lax.fori_loop(0, NUM_CHUNKS, chunk_body, 0)
    pltpu.sync_copy(out_v, out_hbm.at[pl.ds(base, EDGES_PER_WORKER)])


def kernel(node_embeddings, edge_label_index):
    idx = edge_label_index.astype(jnp.int32)
    pad1 = (jnp.arange(E_PAD - N_EDGES, dtype=jnp.int32) * 13) % N_NODES
    pad = jnp.stack([pad1, pad1])
    idx = jnp.concatenate([idx, pad], axis=1)
    src = idx[0]
    dst = idx[1]

    mesh = plsc.VectorSubcoreMesh(core_axis_name="c", subcore_axis_name="s")
    f = pl.kernel(
        _sc_body,
        mesh=mesh,
        compiler_params=pltpu.CompilerParams(needs_layout_passes=False),
        out_type=jax.ShapeDtypeStruct((E_PAD,), jnp.float32),
        scratch_types=[
            pltpu.VMEM((EDGES_PER_WORKER,), jnp.int32),
            pltpu.VMEM((EDGES_PER_WORKER,), jnp.int32),
            pltpu.VMEM((CHUNK, D_FEAT), jnp.float32),
            pltpu.VMEM((CHUNK, D_FEAT), jnp.float32),
            pltpu.VMEM((CHUNK, D_FEAT), jnp.float32),
            pltpu.VMEM((CHUNK, D_FEAT), jnp.float32),
            pltpu.VMEM((EDGES_PER_WORKER,), jnp.float32),
            pltpu.SemaphoreType.DMA,
            pltpu.SemaphoreType.DMA,
        ],
    )
    return f(node_embeddings, src, dst)[:N_EDGES]


# speedup vs baseline: 3.8817x; 1.7841x over previous
import jax
import jax.numpy as jnp
from jax import lax
from jax.experimental import pallas as pl
from jax.experimental.pallas import tpu as pltpu
from jax.experimental.pallas import tpu_sc as plsc

N_NODES = 10000
D_FEAT = 256
N_EDGES = 160000

NUM_CORES = 2
NUM_SUBCORES = 16
NUM_WORKERS = NUM_CORES * NUM_SUBCORES
CHUNK = 40
NUM_CHUNKS = 130
NUM_PAIRS = NUM_CHUNKS // 2
EDGES_PER_WORKER = CHUNK * NUM_CHUNKS
E_PAD = EDGES_PER_WORKER * NUM_WORKERS
LANES = 16


def _sc_body(emb_hbm, src_hbm, dst_hbm, out_hbm,
             idx_s_v, idx_t_v, rows_s_v, rows_t_v, rows_s1, rows_t1, out_v,
             sem, sem1):
    wid = lax.axis_index("s") * NUM_CORES + lax.axis_index("c")
    base = wid * EDGES_PER_WORKER

    pltpu.sync_copy(src_hbm.at[pl.ds(base, EDGES_PER_WORKER)], idx_s_v)
    pltpu.sync_copy(dst_hbm.at[pl.ds(base, EDGES_PER_WORKER)], idx_t_v)

    def fire(ci, rows_s, rows_t, s):
        off = ci * CHUNK
        pltpu.async_copy(emb_hbm.at[idx_s_v.at[pl.ds(off, CHUNK)]], rows_s, s)
        pltpu.async_copy(emb_hbm.at[idx_t_v.at[pl.ds(off, CHUNK)]], rows_t, s)

    def drain(ci, rows_s, rows_t, s):
        off = ci * CHUNK
        pltpu.make_async_copy(
            emb_hbm.at[idx_s_v.at[pl.ds(off, CHUNK)]], rows_s, s).wait()
        pltpu.make_async_copy(
            emb_hbm.at[idx_t_v.at[pl.ds(off, CHUNK)]], rows_t, s).wait()

    lane = lax.iota(jnp.int32, LANES)
    last_lane = lane == (LANES - 1)

    def compute(ci, rows_s, rows_t):
        off = ci * CHUNK

        def edge_body(e, carry2):
            acc = rows_s[e, pl.ds(0, LANES)] * rows_t[e, pl.ds(0, LANES)]
            for j in range(1, D_FEAT // LANES):
                acc = acc + (rows_s[e, pl.ds(j * LANES, LANES)]
                             * rows_t[e, pl.ds(j * LANES, LANES)])
            tot = plsc.cumsum(acc)
            plsc.store_scatter(out_v, [jnp.full((LANES,), off + e, jnp.int32)],
                               tot, mask=last_lane)
            return carry2

        lax.fori_loop(0, CHUNK, edge_body, 0, unroll=4)

    fire(0, rows_s_v, rows_t_v, sem)

    def pair_body(p, carry):
        c0 = 2 * p
        fire(c0 + 1, rows_s1, rows_t1, sem1)
        drain(c0, rows_s_v, rows_t_v, sem)
        compute(c0, rows_s_v, rows_t_v)

        @pl.when(p < NUM_PAIRS - 1)
        def _():
            fire(c0 + 2, rows_s_v, rows_t_v, sem)

        drain(c0 + 1, rows_s1, rows_t1, sem1)
        compute(c0 + 1, rows_s1, rows_t1)
        return carry

    lax.fori_loop(0, NUM_PAIRS, pair_body, 0)
    pltpu.sync_copy(out_v, out_hbm.at[pl.ds(base, EDGES_PER_WORKER)])


def kernel(node_embeddings, edge_label_index):
    idx = edge_label_index.astype(jnp.int32)
    pad1 = (jnp.arange(E_PAD - N_EDGES, dtype=jnp.int32) * 13) % N_NODES
    pad = jnp.stack([pad1, pad1])
    idx = jnp.concatenate([idx, pad], axis=1)
    src = idx[0]
    dst = idx[1]

    mesh = plsc.VectorSubcoreMesh(core_axis_name="c", subcore_axis_name="s")
    f = pl.kernel(
        _sc_body,
        mesh=mesh,
        compiler_params=pltpu.CompilerParams(needs_layout_passes=False),
        out_type=jax.ShapeDtypeStruct((E_PAD,), jnp.float32),
        scratch_types=[
            pltpu.VMEM((EDGES_PER_WORKER,), jnp.int32),
            pltpu.VMEM((EDGES_PER_WORKER,), jnp.int32),
            pltpu.VMEM((CHUNK, D_FEAT), jnp.float32),
            pltpu.VMEM((CHUNK, D_FEAT), jnp.float32),
            pltpu.VMEM((CHUNK, D_FEAT), jnp.float32),
            pltpu.VMEM((CHUNK, D_FEAT), jnp.float32),
            pltpu.VMEM((EDGES_PER_WORKER,), jnp.float32),
            pltpu.SemaphoreType.DMA,
            pltpu.SemaphoreType.DMA,
        ],
    )
    return f(node_embeddings, src, dst)[:N_EDGES]
